# baseline (speedup 1.0000x reference)
import functools

import jax
import jax.numpy as jnp
from jax import lax
from jax.experimental import pallas as pl
from jax.experimental.pallas import tpu as pltpu
from jax.experimental.pallas import tpu_sc as plsc

B, T, D = 4096, 50, 128
NC, NS = 2, 16
NW = NC * NS
BPW = B // NW


def _sc_body(feature_hbm, idx_hbm, table_hbm, out_hbm, idx_v, rows_v, sem):
    wid = lax.axis_index("s") * NC + lax.axis_index("c")
    base = wid * BPW
    pltpu.sync_copy(idx_hbm.at[pl.ds(base, BPW)], idx_v)
    pltpu.async_copy(table_hbm.at[idx_v], rows_v, sem).wait()
    pltpu.sync_copy(rows_v, out_hbm.at[pl.ds(base, BPW), pl.ds(0, D)])
    pltpu.sync_copy(
        feature_hbm.at[pl.ds(base, BPW)],
        out_hbm.at[pl.ds(base, BPW), pl.ds(D, T * D)],
    )


@jax.jit
def _run(feature2d, idx, table):
    mesh = plsc.VectorSubcoreMesh(core_axis_name="c", subcore_axis_name="s")
    fn = functools.partial(
        pl.kernel,
        out_type=jax.ShapeDtypeStruct((B, (T + 1) * D), jnp.float32),
        mesh=mesh,
        scratch_types=[
            pltpu.VMEM((BPW,), jnp.int32),
            pltpu.VMEM((BPW, D), jnp.float32),
            pltpu.SemaphoreType.DMA,
        ],
    )(_sc_body)
    return fn(feature2d, idx, table).reshape(B, T + 1, D)


def kernel(feature, index_value, embedding_table):
    idx = index_value.astype(jnp.int32)
    return _run(feature.reshape(B, T * D), idx, embedding_table)


# speedup vs baseline: 8.4653x; 8.4653x over previous
import functools

import jax
import jax.numpy as jnp
from jax import lax
from jax.experimental import pallas as pl
from jax.experimental.pallas import tpu as pltpu
from jax.experimental.pallas import tpu_sc as plsc

B, T, D = 4096, 50, 128
NC, NS = 2, 16
NW = NC * NS
BPW = B // NW

BLK = 256
GRID = B // BLK


def _sc_gather_body(idx_hbm, table_hbm, emb_hbm, idx_v, rows_v, sem):
    wid = lax.axis_index("s") * NC + lax.axis_index("c")
    base = wid * BPW
    pltpu.sync_copy(idx_hbm.at[pl.ds(base, BPW)], idx_v)
    pltpu.async_copy(table_hbm.at[idx_v], rows_v, sem).wait()
    pltpu.sync_copy(rows_v, emb_hbm.at[pl.ds(base, BPW)])


def _tc_concat_body(emb_ref, feat_ref, out_ref):
    out_ref[:, :D] = emb_ref[...]
    out_ref[:, D:] = feat_ref[...]


@jax.jit
def _run(feature2d, idx, table):
    mesh = plsc.VectorSubcoreMesh(core_axis_name="c", subcore_axis_name="s")
    emb = functools.partial(
        pl.kernel,
        out_type=jax.ShapeDtypeStruct((B, D), jnp.float32),
        mesh=mesh,
        scratch_types=[
            pltpu.VMEM((BPW,), jnp.int32),
            pltpu.VMEM((BPW, D), jnp.float32),
            pltpu.SemaphoreType.DMA,
        ],
    )(_sc_gather_body)(idx, table)

    out2d = pl.pallas_call(
        _tc_concat_body,
        grid=(GRID,),
        in_specs=[
            pl.BlockSpec((BLK, D), lambda i: (i, 0)),
            pl.BlockSpec((BLK, T * D), lambda i: (i, 0)),
        ],
        out_specs=pl.BlockSpec((BLK, (T + 1) * D), lambda i: (i, 0)),
        out_shape=jax.ShapeDtypeStruct((B, (T + 1) * D), jnp.float32),
    )(emb, feature2d)
    return out2d.reshape(B, T + 1, D)


def kernel(feature, index_value, embedding_table):
    idx = index_value.astype(jnp.int32)
    return _run(feature.reshape(B, T * D), idx, embedding_table)


# speedup vs baseline: 15.3273x; 1.8106x over previous
import functools

import jax
import jax.numpy as jnp
from jax import lax
from jax.experimental import pallas as pl
from jax.experimental.pallas import tpu as pltpu
from jax.experimental.pallas import tpu_sc as plsc

B, T, D = 4096, 50, 128
NC, NS = 2, 16
NW = NC * NS
BPW = B // NW

BLK = 256
GRID = B // BLK


def _sc_gather_body(idx_hbm, table_hbm, emb_hbm, idx_v, rows_v, sem):
    wid = lax.axis_index("s") * NC + lax.axis_index("c")
    base = wid * BPW
    pltpu.sync_copy(idx_hbm.at[pl.ds(base, BPW)], idx_v)
    pltpu.async_copy(table_hbm.at[idx_v], rows_v, sem).wait()
    pltpu.sync_copy(rows_v, emb_hbm.at[pl.ds(base, BPW)])


def _tc_concat_body(emb_ref, feat_ref, out_ref):
    out_ref[:, 0, :] = emb_ref[...]
    out_ref[:, 1:, :] = feat_ref[...]


@jax.jit
def _run(feature, idx, table):
    mesh = plsc.VectorSubcoreMesh(core_axis_name="c", subcore_axis_name="s")
    emb = functools.partial(
        pl.kernel,
        out_type=jax.ShapeDtypeStruct((B, D), jnp.float32),
        mesh=mesh,
        scratch_types=[
            pltpu.VMEM((BPW,), jnp.int32),
            pltpu.VMEM((BPW, D), jnp.float32),
            pltpu.SemaphoreType.DMA,
        ],
    )(_sc_gather_body)(idx, table)

    return pl.pallas_call(
        _tc_concat_body,
        grid=(GRID,),
        in_specs=[
            pl.BlockSpec((BLK, D), lambda i: (i, 0)),
            pl.BlockSpec((BLK, T, D), lambda i: (i, 0, 0)),
        ],
        out_specs=pl.BlockSpec((BLK, T + 1, D), lambda i: (i, 0, 0)),
        out_shape=jax.ShapeDtypeStruct((B, T + 1, D), jnp.float32),
    )(emb, feature)


def kernel(feature, index_value, embedding_table):
    idx = index_value.astype(jnp.int32)
    return _run(feature, idx, embedding_table)
